# baseline (speedup 1.0000x reference)
import functools

import jax
import jax.numpy as jnp
from jax import lax
from jax.experimental import pallas as pl
from jax.experimental.pallas import tpu as pltpu
from jax.experimental.pallas import tpu_sc as plsc

D = 64
IPR = 128
CHUNK = 512
K = CHUNK // IPR


@functools.cache
def _build_gather(n_rows):
    info = plsc.get_sparse_core_info()
    nc, ns = info.num_cores, info.num_subcores
    nw = nc * ns
    assert n_rows % (nw * 2 * CHUNK) == 0
    rows_per_w = n_rows // nw
    n_chunks = rows_per_w // CHUNK
    n_pairs = n_chunks // 2
    mesh = plsc.VectorSubcoreMesh(core_axis_name="c", subcore_axis_name="s")

    @functools.partial(
        pl.kernel,
        mesh=mesh,
        compiler_params=pltpu.CompilerParams(use_tc_tiling_on_sc=False),
        out_type=jax.ShapeDtypeStruct((n_rows, D), jnp.float32),
        scratch_types=[
            pltpu.VMEM((K, IPR), jnp.int32),
            pltpu.VMEM((K, IPR), jnp.int32),
            pltpu.VMEM((CHUNK, D), jnp.float32),
            pltpu.VMEM((CHUNK, D), jnp.float32),
            pltpu.SemaphoreType.DMA,
        ],
    )
    def gather(idx_hbm, table_hbm, out_hbm, idx0, idx1, rows0, rows1, sem):
        wid = lax.axis_index("s") * nc + lax.axis_index("c")
        w_row0 = wid * rows_per_w
        w_irow0 = wid * (rows_per_w // IPR)

        idx_bufs = (idx0, idx1)
        row_bufs = (rows0, rows1)

        def store_prev(prows, g):
            pltpu.sync_copy(
                prows, out_hbm.at[pl.ds(w_row0 + (g - 1) * CHUNK, CHUNK)])

        def do_chunk(g, b, store_fn):
            idx_v, rows_v = idx_bufs[b], row_bufs[b]
            prows = row_bufs[1 - b]
            pltpu.sync_copy(idx_hbm.at[pl.ds(w_irow0 + g * K, K)], idx_v)
            handles = [
                pltpu.async_copy(
                    table_hbm.at[idx_v.at[j]],
                    rows_v.at[pl.ds(j * IPR, IPR)],
                    sem,
                )
                for j in range(K)
            ]
            store_fn(prows, g)
            for h in handles:
                h.wait()

        def body(p, carry):
            def store_first(prows, g):
                @pl.when(p > 0)
                def _():
                    store_prev(prows, g)

            do_chunk(2 * p, 0, store_first)
            do_chunk(2 * p + 1, 1, store_prev)
            return carry

        lax.fori_loop(0, n_pairs, body, None)
        pltpu.sync_copy(
            rows1, out_hbm.at[pl.ds(w_row0 + (n_chunks - 1) * CHUNK, CHUNK)])

    return gather


def kernel(POI_id, loc_table, user_table, time_table):
    b, h = POI_id.shape
    idx = POI_id.astype(jnp.int32).reshape(-1, IPR)
    flat = _build_gather(b * h)(idx, loc_table)
    loc_embedded = flat.reshape(b, h, D)
    return (loc_embedded, time_table, user_table)


# speedup vs baseline: 1.0332x; 1.0332x over previous
import functools

import jax
import jax.numpy as jnp
from jax import lax
from jax.experimental import pallas as pl
from jax.experimental.pallas import tpu as pltpu
from jax.experimental.pallas import tpu_sc as plsc

D = 64
IPR = 128
CHUNK = 640
K = CHUNK // IPR


@functools.cache
def _build_gather(n_rows):
    info = plsc.get_sparse_core_info()
    nc, ns = info.num_cores, info.num_subcores
    nw = nc * ns
    assert n_rows % (nw * 2 * CHUNK) == 0
    rows_per_w = n_rows // nw
    n_chunks = rows_per_w // CHUNK
    n_pairs = n_chunks // 2
    mesh = plsc.VectorSubcoreMesh(core_axis_name="c", subcore_axis_name="s")

    @functools.partial(
        pl.kernel,
        mesh=mesh,
        compiler_params=pltpu.CompilerParams(use_tc_tiling_on_sc=False),
        out_type=jax.ShapeDtypeStruct((n_rows, D), jnp.float32),
        scratch_types=[
            pltpu.VMEM((K, IPR), jnp.int32),
            pltpu.VMEM((K, IPR), jnp.int32),
            pltpu.VMEM((CHUNK, D), jnp.float32),
            pltpu.VMEM((CHUNK, D), jnp.float32),
            pltpu.SemaphoreType.DMA,
            pltpu.SemaphoreType.DMA,
            pltpu.SemaphoreType.DMA,
            pltpu.SemaphoreType.DMA,
            pltpu.SemaphoreType.DMA,
            pltpu.SemaphoreType.DMA,
        ],
    )
    def gather(idx_hbm, table_hbm, out_hbm, idx0, idx1, rows0, rows1,
               gsem0, gsem1, ssem0, ssem1, isem0, isem1):
        wid = lax.axis_index("s") * nc + lax.axis_index("c")
        w_row0 = wid * rows_per_w
        w_irow0 = wid * (rows_per_w // IPR)

        idx_bufs = (idx0, idx1)
        row_bufs = (rows0, rows1)
        gsems = (gsem0, gsem1)
        ssems = (ssem0, ssem1)
        isems = (isem0, isem1)

        def fire_idx(g, b):
            pltpu.async_copy(
                idx_hbm.at[pl.ds(w_irow0 + g * K, K)], idx_bufs[b], isems[b])

        def drain_rows(sem, buf):
            pltpu.make_async_copy(out_hbm.at[pl.ds(w_row0, CHUNK)], buf,
                                  sem).wait()

        def drain_store(b):
            pltpu.make_async_copy(
                row_bufs[b], out_hbm.at[pl.ds(w_row0, CHUNK)], ssems[b]).wait()

        def stage(p, g, b):
            o = 1 - b
            @pl.when(p > 0)
            def _():
                drain_store(b)
            drain_rows(isems[b], idx_bufs[b])
            for j in range(K):
                pltpu.async_copy(
                    table_hbm.at[idx_bufs[b].at[j]],
                    row_bufs[b].at[pl.ds(j * IPR, IPR)],
                    gsems[b],
                )
            if b == 0:
                @pl.when(p > 0)
                def _():
                    drain_rows(gsems[o], row_bufs[o])
                    fire_idx(g + 1, o)
                    pltpu.async_copy(
                        row_bufs[o],
                        out_hbm.at[pl.ds(w_row0 + (g - 1) * CHUNK, CHUNK)],
                        ssems[o])

                @pl.when(p == 0)
                def _():
                    fire_idx(g + 1, o)
            else:
                drain_rows(gsems[o], row_bufs[o])

                @pl.when(p < n_pairs - 1)
                def _():
                    fire_idx(g + 1, o)

                pltpu.async_copy(
                    row_bufs[o],
                    out_hbm.at[pl.ds(w_row0 + (g - 1) * CHUNK, CHUNK)],
                    ssems[o])

        def body(p, carry):
            stage(p, 2 * p, 0)
            stage(p, 2 * p + 1, 1)
            return carry

        fire_idx(0, 0)
        lax.fori_loop(0, n_pairs, body, None)
        drain_rows(gsems[1], rows1)
        pltpu.async_copy(
            rows1, out_hbm.at[pl.ds(w_row0 + (n_chunks - 1) * CHUNK, CHUNK)],
            ssems[1])
        drain_store(0)
        drain_store(1)

    return gather


def kernel(POI_id, loc_table, user_table, time_table):
    b, h = POI_id.shape
    idx = POI_id.astype(jnp.int32).reshape(-1, IPR)
    flat = _build_gather(b * h)(idx, loc_table)
    loc_embedded = flat.reshape(b, h, D)
    return (loc_embedded, time_table, user_table)
